# baseline (speedup 1.0000x reference)
import functools

import jax
import jax.numpy as jnp
from jax import lax
from jax.experimental import pallas as pl
from jax.experimental.pallas import tpu as pltpu
from jax.experimental.pallas import tpu_sc as plsc

NN = 100000
EE = 1600000
NW = 32
BATCH = 128
STEPS = 400
OUTER = 25
G = 16
C0 = 34
C1 = 16
EPAD = STEPS * BATCH * NW
NPAD = 100352
ZCH = NPAD // (16 * BATCH)
BN = 2048
BW = 256
NPW = NPAD * 16 // 128


def _gelu(x):
    return 0.5 * x * (1.0 + lax.erf(x * 0.7071067811865476))


def _ln(x, g, b, eps=1e-5):
    m = jnp.mean(x, axis=-1, keepdims=True)
    v = jnp.mean((x - m) ** 2, axis=-1, keepdims=True)
    return (x - m) / jnp.sqrt(v + eps) * g + b


@functools.lru_cache(maxsize=None)
def _get_sc_degree():
    mesh = plsc.VectorSubcoreMesh(core_axis_name="c", subcore_axis_name="s")
    return pl.kernel(
        _sc_degree_body,
        out_type=[jax.ShapeDtypeStruct((NPAD,), jnp.float32),
                  jax.ShapeDtypeStruct((NPAD,), jnp.float32)],
        mesh=mesh,
        compiler_params=pltpu.CompilerParams(use_tc_tiling_on_sc=False),
        scratch_types=[
            pltpu.VMEM_SHARED((NPAD,), jnp.float32),
            pltpu.VMEM((G, BATCH), jnp.int32),
            pltpu.VMEM((BATCH,), jnp.float32),
            pltpu.VMEM((BATCH,), jnp.float32),
            pltpu.VMEM((392,), jnp.float32),
        ],
    )


def _sc_degree_body(dst_hbm, out0_hbm, out1_hbm, acc, didx, ones_v, zero_v, bounce):
    c = lax.axis_index("c")
    s = lax.axis_index("s")
    wid = c * 16 + s
    for r in range(BATCH // 16):
        ones_v[pl.ds(r * 16, 16)] = jnp.ones((16,), jnp.float32)
        zero_v[pl.ds(r * 16, 16)] = jnp.zeros((16,), jnp.float32)

    def zb(i, carry):
        pltpu.sync_copy(zero_v, acc.at[pl.ds((s * ZCH + i) * BATCH, BATCH)])
        return carry

    lax.fori_loop(0, ZCH, zb, 0)
    plsc.subcore_barrier()

    def outer(g, carry):
        pltpu.sync_copy(dst_hbm.at[pl.ds(wid * STEPS + g * G, G)], didx)

        def body(j, carry2):
            pltpu.sync_copy(ones_v, acc.at[didx.at[j]], add=True)
            return carry2

        return lax.fori_loop(0, G, body, carry)

    lax.fori_loop(0, OUTER, outer, 0)
    plsc.subcore_barrier()

    def ocp(i, carry):
        pltpu.sync_copy(acc.at[pl.ds(s * 6272 + i * 392, 392)], bounce)

        @pl.when(c == 0)
        def _():
            pltpu.sync_copy(bounce, out0_hbm.at[pl.ds(s * 6272 + i * 392, 392)])

        @pl.when(c == 1)
        def _():
            pltpu.sync_copy(bounce, out1_hbm.at[pl.ds(s * 6272 + i * 392, 392)])

        return carry

    lax.fori_loop(0, 16, ocp, 0)


@functools.lru_cache(maxsize=None)
def _get_sc_round():
    mesh = plsc.VectorSubcoreMesh(core_axis_name="c", subcore_axis_name="s")
    return pl.kernel(
        _sc_round_body,
        out_type=jax.ShapeDtypeStruct((2, NPW, 128), jnp.float32),
        mesh=mesh,
        compiler_params=pltpu.CompilerParams(use_tc_tiling_on_sc=False),
        scratch_types=[
            pltpu.VMEM_SHARED((NPAD, 16), jnp.float32),
            pltpu.VMEM((2, G, BATCH), jnp.int32),
            pltpu.VMEM((2, G, BATCH), jnp.int32),
            pltpu.VMEM((BATCH, 16), jnp.float32),
            pltpu.VMEM((BATCH, 16), jnp.float32),
            pltpu.VMEM((BATCH, 16), jnp.float32),
            pltpu.VMEM((BATCH, 16), jnp.float32),
            pltpu.VMEM((BATCH, 16), jnp.float32),
            pltpu.VMEM((16, 128), jnp.float32),
            pltpu.SemaphoreType.DMA,
            pltpu.SemaphoreType.DMA,
            pltpu.SemaphoreType.DMA,
            pltpu.SemaphoreType.DMA,
            pltpu.SemaphoreType.DMA,
            pltpu.SemaphoreType.DMA,
        ],
    )


def _sc_round_body(tab_hbm, src_hbm, dst_hbm, out_hbm, acc, sidx, didx, zbuf,
                   rows_a, rows_b, bnc_a, bnc_b, bounce2, sem_a, sem_b, sem_si,
                   sem_di, sem_oa, sem_ob):
    c = lax.axis_index("c")
    s = lax.axis_index("s")
    for r in range(BATCH):
        zbuf[r] = jnp.zeros((16,), jnp.float32)

    def zb(i, carry):
        pltpu.sync_copy(zbuf, acc.at[pl.ds((s * ZCH + i) * BATCH, BATCH)])
        return carry

    lax.fori_loop(0, ZCH, zb, 0)
    plsc.subcore_barrier()

    bufs = (rows_a, rows_b)
    sems = (sem_a, sem_b)
    nch = jnp.where(c == 0, C0, C1)
    base = jnp.where(c == 0, s * C0, 16 * C0 + s * C1)
    pltpu.sync_copy(src_hbm.at[pl.ds(base * G, G)], sidx.at[0])
    pltpu.sync_copy(dst_hbm.at[pl.ds(base * G, G)], didx.at[0])

    def outer(g, carry):
        p = g % 2

        @pl.when(g + 1 < nch)
        def _():
            row0 = (base + g + 1) * G
            pltpu.async_copy(src_hbm.at[pl.ds(row0, G)], sidx.at[1 - p], sem_si)
            pltpu.async_copy(dst_hbm.at[pl.ds(row0, G)], didx.at[1 - p], sem_di)

        pend = pltpu.async_copy(tab_hbm.at[sidx.at[p, 0]], bufs[0], sems[0])
        for j in range(G):
            if j + 1 < G:
                nxt = pltpu.async_copy(
                    tab_hbm.at[sidx.at[p, j + 1]], bufs[(j + 1) % 2], sems[(j + 1) % 2])
            pend.wait()
            pltpu.sync_copy(bufs[j % 2], acc.at[didx.at[p, j]], add=True)
            if j + 1 < G:
                pend = nxt

        @pl.when(g + 1 < nch)
        def _():
            pltpu.make_async_copy(src_hbm.at[pl.ds(0, G)], sidx.at[1 - p], sem_si).wait()
            pltpu.make_async_copy(dst_hbm.at[pl.ds(0, G)], didx.at[1 - p], sem_di).wait()

        return carry

    lax.fori_loop(0, nch, outer, 0)
    plsc.subcore_barrier()

    def emit(bnc, i):
        for r in range(BATCH):
            bounce2[r // 8, pl.ds((r % 8) * 16, 16)] = bnc[r]
        pltpu.sync_copy(bounce2, out_hbm.at[c, pl.ds(s * 784 + i * 16, 16)])

    def arow(i):
        return acc.at[pl.ds(s * 6272 + i * BATCH, BATCH)]

    pltpu.sync_copy(arow(0), bnc_a)

    def ocp(k, carry):
        hb = pltpu.async_copy(arow(2 * k + 1), bnc_b, sem_ob)
        emit(bnc_a, 2 * k)
        hb.wait()
        ha = pltpu.async_copy(arow(2 * k + 2), bnc_a, sem_oa)
        emit(bnc_b, 2 * k + 1)
        ha.wait()
        return carry

    lax.fori_loop(0, ZCH // 2, ocp, 0)
    emit(bnc_a, ZCH - 1)


def _enc_body(x0_r, x1_r,
              w01, b01, g01, bb01, w02, b02, rg0, rb0,
              w11, b11, g11, bb11, w12, b12, rg1, rb1,
              cw0, cw1, cb, u_r):
    def enc(x, W1, b1, g1, bb1, W2, b2, rg, rb):
        h = jnp.dot(x, W1[...], preferred_element_type=jnp.float32) + b1[...]
        h = _ln(h, g1[...], bb1[...])
        h = _gelu(h)
        h2 = jnp.dot(h, W2[...], preferred_element_type=jnp.float32) + b2[...]
        y = h2 + h2
        return _ln(y, rg[...], rb[...])

    z0 = enc(x0_r[...], w01, b01, g01, bb01, w02, b02, rg0, rb0)
    z1 = enc(x1_r[...], w11, b11, g11, bb11, w12, b12, rg1, rb1)
    u = (jnp.dot(z0, cw0[...], preferred_element_type=jnp.float32)
         + jnp.dot(z1, cw1[...], preferred_element_type=jnp.float32) + cb[...])
    u_r[...] = u


def _scale_body(u_r, dga_r, dgb_r, t1_r):
    dinv = lax.rsqrt(dga_r[...] + dgb_r[...] + 1.0)[:, None]
    t1_r[...] = u_r[...] * dinv


def _mid_body(s1_r, t1_r, dga8_r, dgb8_r, e_r, s_r, gt, bt, cbt, zw_r, t2w_r):
    dinv8 = lax.rsqrt(dga8_r[...] + dgb8_r[...] + 1.0)
    dinv_w = jnp.dot(dinv8, e_r[...], preferred_element_type=jnp.float32)
    s1 = s1_r[...]
    zp = (s1[0] + s1[1] + t1_r[...]) * dinv_w + cbt[...]
    sm = s_r[...]
    m = jnp.dot(zp, sm, preferred_element_type=jnp.float32)
    v = jnp.dot(zp * zp, sm, preferred_element_type=jnp.float32) - m * m
    z = (zp - m) * lax.rsqrt(v + 1e-5) * gt[...] + bt[...]
    zw_r[...] = z
    t2w_r[...] = z * dinv_w


def _dom_body(z_r, w1, b1, w2, b2, dom_r):
    h = _gelu(jnp.dot(z_r[...], w1[...], preferred_element_type=jnp.float32) + b1[...])
    dom_r[...] = jnp.dot(h, w2[...], preferred_element_type=jnp.float32) + b2[...]


def _dec_body(s2_r, t2_r, dga8_r, dgb8_r, e_r, d0wb, d0bb, d1wb, d1bb, rec0_r, rec1_r):
    dinv8 = lax.rsqrt(dga8_r[...] + dgb8_r[...] + 1.0)
    dinv_w = jnp.dot(dinv8, e_r[...], preferred_element_type=jnp.float32)
    s2 = s2_r[...]
    agg = (s2[0] + s2[1] + t2_r[...]) * dinv_w
    r0 = jnp.dot(agg, d0wb[...], preferred_element_type=jnp.float32) + d0bb[...]
    r1 = jnp.dot(agg, d1wb[...], preferred_element_type=jnp.float32) + d1bb[...]
    rec0_r[...] = r0.reshape(BN, 128)
    rec1_r[...] = r1.reshape(BN, 128)


def _full(shape):
    return pl.BlockSpec(shape, lambda i: tuple(0 for _ in shape))


def _rows(width):
    return pl.BlockSpec((BN, width), lambda i: (i, 0))


_GRID = -(-NN // BN)
_DEG1 = pl.BlockSpec((BN,), lambda i: (i,))
_DEG8 = pl.BlockSpec((BW, 8), lambda i: (i, 0))
_WROWS = pl.BlockSpec((BW, 128), lambda i: (i, 0))
_WPART = pl.BlockSpec((2, BW, 128), lambda i: (0, i, 0))


def kernel(x0, x1, edge_index, enc0_W1, enc0_b1, enc0_g1, enc0_bb1, enc0_W2,
           enc0_b2, enc0_rg, enc0_rb, enc1_W1, enc1_b1, enc1_g1, enc1_bb1,
           enc1_W2, enc1_b2, enc1_rg, enc1_rb, comb_W, comb_b, comb_g,
           comb_bb, dec0_W, dec0_b, dec1_W, dec1_b, clf_W1, clf_b1, clf_W2,
           clf_b2):
    f32 = jnp.float32
    src = jnp.concatenate(
        [edge_index[0], jnp.zeros((EPAD - EE,), jnp.int32)]).reshape(
            NW * STEPS, BATCH)
    dst = jnp.concatenate(
        [edge_index[1], jnp.full((EPAD - EE,), NN, jnp.int32)]).reshape(
            NW * STEPS, BATCH)

    dga, dgb = _get_sc_degree()(dst)
    dga8 = dga.reshape(NPW, 8)
    dgb8 = dgb.reshape(NPW, 8)

    eye8 = jnp.eye(8, dtype=f32)
    e_mat = jnp.repeat(eye8, 16, axis=1)
    s_mat = jnp.kron(eye8, jnp.full((16, 16), 1.0 / 16.0, f32))
    d0wb = jnp.kron(eye8, dec0_W)
    d1wb = jnp.kron(eye8, dec1_W)
    d0bb = jnp.tile(dec0_b, 8)[None]
    d1bb = jnp.tile(dec1_b, 8)[None]
    gt = jnp.tile(comb_g, 8)[None]
    bt = jnp.tile(comb_bb, 8)[None]
    cbt = jnp.tile(comb_b, 8)[None]

    row = lambda a: a[None, :]
    u = pl.pallas_call(
        _enc_body,
        grid=(_GRID,),
        in_specs=[
            _rows(128), _rows(128),
            _full((128, 64)), _full((1, 64)), _full((1, 64)), _full((1, 64)),
            _full((64, 16)), _full((1, 16)), _full((1, 16)), _full((1, 16)),
            _full((128, 64)), _full((1, 64)), _full((1, 64)), _full((1, 64)),
            _full((64, 16)), _full((1, 16)), _full((1, 16)), _full((1, 16)),
            _full((16, 16)), _full((16, 16)), _full((1, 16)),
        ],
        out_specs=_rows(16),
        out_shape=jax.ShapeDtypeStruct((NN, 16), f32),
    )(x0, x1,
      enc0_W1, row(enc0_b1), row(enc0_g1), row(enc0_bb1),
      enc0_W2, row(enc0_b2), row(enc0_rg), row(enc0_rb),
      enc1_W1, row(enc1_b1), row(enc1_g1), row(enc1_bb1),
      enc1_W2, row(enc1_b2), row(enc1_rg), row(enc1_rb),
      comb_W[:16], comb_W[16:], row(comb_b))

    t1 = pl.pallas_call(
        _scale_body,
        grid=(_GRID,),
        in_specs=[_rows(16), _DEG1, _DEG1],
        out_specs=_rows(16),
        out_shape=jax.ShapeDtypeStruct((NPAD, 16), f32),
    )(u, dga, dgb)

    s1 = _get_sc_round()(t1, src, dst)
    t1w = t1.reshape(NPW, 128)

    zw, t2w = pl.pallas_call(
        _mid_body,
        grid=(_GRID,),
        in_specs=[
            _WPART, _WROWS, _DEG8, _DEG8,
            _full((8, 128)), _full((128, 128)),
            _full((1, 128)), _full((1, 128)), _full((1, 128)),
        ],
        out_specs=[_WROWS, _WROWS],
        out_shape=[
            jax.ShapeDtypeStruct((NPW, 128), f32),
            jax.ShapeDtypeStruct((NPW, 128), f32),
        ],
    )(s1, t1w, dga8, dgb8, e_mat, s_mat, gt, bt, cbt)

    t2 = t2w.reshape(NPAD, 16)
    s2 = _get_sc_round()(t2, src, dst)

    z = zw.reshape(NPAD, 16)[:NN]
    dom = pl.pallas_call(
        _dom_body,
        grid=(_GRID,),
        in_specs=[
            _rows(16),
            _full((16, 64)), _full((1, 64)),
            _full((64, 8)), _full((1, 8)),
        ],
        out_specs=_rows(8),
        out_shape=jax.ShapeDtypeStruct((NN, 8), f32),
    )(z, clf_W1, row(clf_b1), clf_W2, row(clf_b2))

    rec0, rec1 = pl.pallas_call(
        _dec_body,
        grid=(_GRID,),
        in_specs=[
            _WPART, _WROWS, _DEG8, _DEG8,
            _full((8, 128)),
            _full((128, 1024)), _full((1, 1024)),
            _full((128, 1024)), _full((1, 1024)),
        ],
        out_specs=[_rows(128), _rows(128)],
        out_shape=[
            jax.ShapeDtypeStruct((NN, 128), f32),
            jax.ShapeDtypeStruct((NN, 128), f32),
        ],
    )(s2, t2w, dga8, dgb8, e_mat, d0wb, d0bb, d1wb, d1bb)

    return z, rec0, rec1, dom


# speedup vs baseline: 1.0149x; 1.0149x over previous
import functools

import jax
import jax.numpy as jnp
from jax import lax
from jax.experimental import pallas as pl
from jax.experimental.pallas import tpu as pltpu
from jax.experimental.pallas import tpu_sc as plsc

NN = 100000
EE = 1600000
NW = 32
BATCH = 128
STEPS = 400
OUTER = 25
G = 16
C0 = 38
C1 = 12
EPAD = STEPS * BATCH * NW
NPAD = 100352
ZCH = NPAD // (16 * BATCH)
BN = 2048
BW = 256
NPW = NPAD * 16 // 128


def _gelu(x):
    return 0.5 * x * (1.0 + lax.erf(x * 0.7071067811865476))


def _ln(x, g, b, eps=1e-5):
    m = jnp.mean(x, axis=-1, keepdims=True)
    v = jnp.mean((x - m) ** 2, axis=-1, keepdims=True)
    return (x - m) / jnp.sqrt(v + eps) * g + b


@functools.lru_cache(maxsize=None)
def _get_sc_degree():
    mesh = plsc.VectorSubcoreMesh(core_axis_name="c", subcore_axis_name="s")
    return pl.kernel(
        _sc_degree_body,
        out_type=[jax.ShapeDtypeStruct((NPAD,), jnp.float32),
                  jax.ShapeDtypeStruct((NPAD,), jnp.float32)],
        mesh=mesh,
        compiler_params=pltpu.CompilerParams(use_tc_tiling_on_sc=False),
        scratch_types=[
            pltpu.VMEM_SHARED((NPAD,), jnp.float32),
            pltpu.VMEM((G, BATCH), jnp.int32),
            pltpu.VMEM((BATCH,), jnp.float32),
            pltpu.VMEM((BATCH,), jnp.float32),
            pltpu.VMEM((392,), jnp.float32),
        ],
    )


def _sc_degree_body(dst_hbm, out0_hbm, out1_hbm, acc, didx, ones_v, zero_v, bounce):
    c = lax.axis_index("c")
    s = lax.axis_index("s")
    wid = c * 16 + s
    for r in range(BATCH // 16):
        ones_v[pl.ds(r * 16, 16)] = jnp.ones((16,), jnp.float32)
        zero_v[pl.ds(r * 16, 16)] = jnp.zeros((16,), jnp.float32)

    def zb(i, carry):
        pltpu.sync_copy(zero_v, acc.at[pl.ds((s * ZCH + i) * BATCH, BATCH)])
        return carry

    lax.fori_loop(0, ZCH, zb, 0)
    plsc.subcore_barrier()

    def outer(g, carry):
        pltpu.sync_copy(dst_hbm.at[pl.ds(wid * STEPS + g * G, G)], didx)

        def body(j, carry2):
            pltpu.sync_copy(ones_v, acc.at[didx.at[j]], add=True)
            return carry2

        return lax.fori_loop(0, G, body, carry)

    lax.fori_loop(0, OUTER, outer, 0)
    plsc.subcore_barrier()

    def ocp(i, carry):
        pltpu.sync_copy(acc.at[pl.ds(s * 6272 + i * 392, 392)], bounce)

        @pl.when(c == 0)
        def _():
            pltpu.sync_copy(bounce, out0_hbm.at[pl.ds(s * 6272 + i * 392, 392)])

        @pl.when(c == 1)
        def _():
            pltpu.sync_copy(bounce, out1_hbm.at[pl.ds(s * 6272 + i * 392, 392)])

        return carry

    lax.fori_loop(0, 16, ocp, 0)


@functools.lru_cache(maxsize=None)
def _get_sc_round():
    mesh = plsc.VectorSubcoreMesh(core_axis_name="c", subcore_axis_name="s")
    return pl.kernel(
        _sc_round_body,
        out_type=jax.ShapeDtypeStruct((2, NPW, 128), jnp.float32),
        mesh=mesh,
        compiler_params=pltpu.CompilerParams(use_tc_tiling_on_sc=False),
        scratch_types=[
            pltpu.VMEM_SHARED((NPAD, 16), jnp.float32),
            pltpu.VMEM((2, G, BATCH), jnp.int32),
            pltpu.VMEM((2, G, BATCH), jnp.int32),
            pltpu.VMEM((BATCH, 16), jnp.float32),
            pltpu.VMEM((BATCH, 16), jnp.float32),
            pltpu.VMEM((BATCH, 16), jnp.float32),
            pltpu.VMEM((BATCH, 16), jnp.float32),
            pltpu.VMEM((BATCH, 16), jnp.float32),
            pltpu.VMEM((16, 128), jnp.float32),
            pltpu.SemaphoreType.DMA,
            pltpu.SemaphoreType.DMA,
            pltpu.SemaphoreType.DMA,
            pltpu.SemaphoreType.DMA,
            pltpu.SemaphoreType.DMA,
            pltpu.SemaphoreType.DMA,
        ],
    )


def _sc_round_body(tab_hbm, src_hbm, dst_hbm, out_hbm, acc, sidx, didx, zbuf,
                   rows_a, rows_b, bnc_a, bnc_b, bounce2, sem_a, sem_b, sem_si,
                   sem_di, sem_oa, sem_ob):
    c = lax.axis_index("c")
    s = lax.axis_index("s")
    for r in range(BATCH):
        zbuf[r] = jnp.zeros((16,), jnp.float32)

    def zb(i, carry):
        pltpu.sync_copy(zbuf, acc.at[pl.ds((s * ZCH + i) * BATCH, BATCH)])
        return carry

    lax.fori_loop(0, ZCH, zb, 0)
    plsc.subcore_barrier()

    bufs = (rows_a, rows_b)
    sems = (sem_a, sem_b)
    nch = jnp.where(c == 0, C0, C1)
    base = jnp.where(c == 0, s * C0, 16 * C0 + s * C1)
    pltpu.sync_copy(src_hbm.at[pl.ds(base * G, G)], sidx.at[0])
    pltpu.sync_copy(dst_hbm.at[pl.ds(base * G, G)], didx.at[0])

    def outer(g, carry):
        p = g % 2

        @pl.when(g + 1 < nch)
        def _():
            row0 = (base + g + 1) * G
            pltpu.async_copy(src_hbm.at[pl.ds(row0, G)], sidx.at[1 - p], sem_si)
            pltpu.async_copy(dst_hbm.at[pl.ds(row0, G)], didx.at[1 - p], sem_di)

        pend = pltpu.async_copy(tab_hbm.at[sidx.at[p, 0]], bufs[0], sems[0])
        for j in range(G):
            if j + 1 < G:
                nxt = pltpu.async_copy(
                    tab_hbm.at[sidx.at[p, j + 1]], bufs[(j + 1) % 2], sems[(j + 1) % 2])
            pend.wait()
            pltpu.sync_copy(bufs[j % 2], acc.at[didx.at[p, j]], add=True)
            if j + 1 < G:
                pend = nxt

        @pl.when(g + 1 < nch)
        def _():
            pltpu.make_async_copy(src_hbm.at[pl.ds(0, G)], sidx.at[1 - p], sem_si).wait()
            pltpu.make_async_copy(dst_hbm.at[pl.ds(0, G)], didx.at[1 - p], sem_di).wait()

        return carry

    lax.fori_loop(0, nch, outer, 0)
    plsc.subcore_barrier()

    def emit(bnc, i):
        for r in range(BATCH):
            bounce2[r // 8, pl.ds((r % 8) * 16, 16)] = bnc[r]
        pltpu.sync_copy(bounce2, out_hbm.at[c, pl.ds(s * 784 + i * 16, 16)])

    def arow(i):
        return acc.at[pl.ds(s * 6272 + i * BATCH, BATCH)]

    pltpu.sync_copy(arow(0), bnc_a)

    def ocp(k, carry):
        hb = pltpu.async_copy(arow(2 * k + 1), bnc_b, sem_ob)
        emit(bnc_a, 2 * k)
        hb.wait()
        ha = pltpu.async_copy(arow(2 * k + 2), bnc_a, sem_oa)
        emit(bnc_b, 2 * k + 1)
        ha.wait()
        return carry

    lax.fori_loop(0, ZCH // 2, ocp, 0)
    emit(bnc_a, ZCH - 1)


def _enc_body(x0_r, x1_r,
              w01, b01, g01, bb01, w02, b02, rg0, rb0,
              w11, b11, g11, bb11, w12, b12, rg1, rb1,
              cw0, cw1, cb, u_r):
    def enc(x, W1, b1, g1, bb1, W2, b2, rg, rb):
        h = jnp.dot(x, W1[...], preferred_element_type=jnp.float32) + b1[...]
        h = _ln(h, g1[...], bb1[...])
        h = _gelu(h)
        h2 = jnp.dot(h, W2[...], preferred_element_type=jnp.float32) + b2[...]
        y = h2 + h2
        return _ln(y, rg[...], rb[...])

    z0 = enc(x0_r[...], w01, b01, g01, bb01, w02, b02, rg0, rb0)
    z1 = enc(x1_r[...], w11, b11, g11, bb11, w12, b12, rg1, rb1)
    u = (jnp.dot(z0, cw0[...], preferred_element_type=jnp.float32)
         + jnp.dot(z1, cw1[...], preferred_element_type=jnp.float32) + cb[...])
    u_r[...] = u


def _scale_body(u_r, dga_r, dgb_r, t1_r):
    dinv = lax.rsqrt(dga_r[...] + dgb_r[...] + 1.0)[:, None]
    t1_r[...] = u_r[...] * dinv


def _mid_body(s1_r, t1_r, dga8_r, dgb8_r, e_r, s_r, gt, bt, cbt, zw_r, t2w_r):
    dinv8 = lax.rsqrt(dga8_r[...] + dgb8_r[...] + 1.0)
    dinv_w = jnp.dot(dinv8, e_r[...], preferred_element_type=jnp.float32)
    s1 = s1_r[...]
    zp = (s1[0] + s1[1] + t1_r[...]) * dinv_w + cbt[...]
    sm = s_r[...]
    m = jnp.dot(zp, sm, preferred_element_type=jnp.float32)
    v = jnp.dot(zp * zp, sm, preferred_element_type=jnp.float32) - m * m
    z = (zp - m) * lax.rsqrt(v + 1e-5) * gt[...] + bt[...]
    zw_r[...] = z
    t2w_r[...] = z * dinv_w


def _dom_body(z_r, w1, b1, w2, b2, dom_r):
    h = _gelu(jnp.dot(z_r[...], w1[...], preferred_element_type=jnp.float32) + b1[...])
    dom_r[...] = jnp.dot(h, w2[...], preferred_element_type=jnp.float32) + b2[...]


def _dec_body(s2_r, t2_r, dga8_r, dgb8_r, e_r, d0wb, d0bb, d1wb, d1bb, rec0_r, rec1_r):
    dinv8 = lax.rsqrt(dga8_r[...] + dgb8_r[...] + 1.0)
    dinv_w = jnp.dot(dinv8, e_r[...], preferred_element_type=jnp.float32)
    s2 = s2_r[...]
    agg = (s2[0] + s2[1] + t2_r[...]) * dinv_w
    r0 = jnp.dot(agg, d0wb[...], preferred_element_type=jnp.float32) + d0bb[...]
    r1 = jnp.dot(agg, d1wb[...], preferred_element_type=jnp.float32) + d1bb[...]
    rec0_r[...] = r0.reshape(BN, 128)
    rec1_r[...] = r1.reshape(BN, 128)


def _full(shape):
    return pl.BlockSpec(shape, lambda i: tuple(0 for _ in shape))


def _rows(width):
    return pl.BlockSpec((BN, width), lambda i: (i, 0))


_GRID = -(-NN // BN)
_DEG1 = pl.BlockSpec((BN,), lambda i: (i,))
_DEG8 = pl.BlockSpec((BW, 8), lambda i: (i, 0))
_WROWS = pl.BlockSpec((BW, 128), lambda i: (i, 0))
_WPART = pl.BlockSpec((2, BW, 128), lambda i: (0, i, 0))


def kernel(x0, x1, edge_index, enc0_W1, enc0_b1, enc0_g1, enc0_bb1, enc0_W2,
           enc0_b2, enc0_rg, enc0_rb, enc1_W1, enc1_b1, enc1_g1, enc1_bb1,
           enc1_W2, enc1_b2, enc1_rg, enc1_rb, comb_W, comb_b, comb_g,
           comb_bb, dec0_W, dec0_b, dec1_W, dec1_b, clf_W1, clf_b1, clf_W2,
           clf_b2):
    f32 = jnp.float32
    src = jnp.concatenate(
        [edge_index[0], jnp.zeros((EPAD - EE,), jnp.int32)]).reshape(
            NW * STEPS, BATCH)
    dst = jnp.concatenate(
        [edge_index[1], jnp.full((EPAD - EE,), NN, jnp.int32)]).reshape(
            NW * STEPS, BATCH)

    dga, dgb = _get_sc_degree()(dst)
    dga8 = dga.reshape(NPW, 8)
    dgb8 = dgb.reshape(NPW, 8)

    eye8 = jnp.eye(8, dtype=f32)
    e_mat = jnp.repeat(eye8, 16, axis=1)
    s_mat = jnp.kron(eye8, jnp.full((16, 16), 1.0 / 16.0, f32))
    d0wb = jnp.kron(eye8, dec0_W)
    d1wb = jnp.kron(eye8, dec1_W)
    d0bb = jnp.tile(dec0_b, 8)[None]
    d1bb = jnp.tile(dec1_b, 8)[None]
    gt = jnp.tile(comb_g, 8)[None]
    bt = jnp.tile(comb_bb, 8)[None]
    cbt = jnp.tile(comb_b, 8)[None]

    row = lambda a: a[None, :]
    u = pl.pallas_call(
        _enc_body,
        grid=(_GRID,),
        in_specs=[
            _rows(128), _rows(128),
            _full((128, 64)), _full((1, 64)), _full((1, 64)), _full((1, 64)),
            _full((64, 16)), _full((1, 16)), _full((1, 16)), _full((1, 16)),
            _full((128, 64)), _full((1, 64)), _full((1, 64)), _full((1, 64)),
            _full((64, 16)), _full((1, 16)), _full((1, 16)), _full((1, 16)),
            _full((16, 16)), _full((16, 16)), _full((1, 16)),
        ],
        out_specs=_rows(16),
        out_shape=jax.ShapeDtypeStruct((NN, 16), f32),
    )(x0, x1,
      enc0_W1, row(enc0_b1), row(enc0_g1), row(enc0_bb1),
      enc0_W2, row(enc0_b2), row(enc0_rg), row(enc0_rb),
      enc1_W1, row(enc1_b1), row(enc1_g1), row(enc1_bb1),
      enc1_W2, row(enc1_b2), row(enc1_rg), row(enc1_rb),
      comb_W[:16], comb_W[16:], row(comb_b))

    t1 = pl.pallas_call(
        _scale_body,
        grid=(_GRID,),
        in_specs=[_rows(16), _DEG1, _DEG1],
        out_specs=_rows(16),
        out_shape=jax.ShapeDtypeStruct((NPAD, 16), f32),
    )(u, dga, dgb)

    s1 = _get_sc_round()(t1, src, dst)
    t1w = t1.reshape(NPW, 128)

    zw, t2w = pl.pallas_call(
        _mid_body,
        grid=(_GRID,),
        in_specs=[
            _WPART, _WROWS, _DEG8, _DEG8,
            _full((8, 128)), _full((128, 128)),
            _full((1, 128)), _full((1, 128)), _full((1, 128)),
        ],
        out_specs=[_WROWS, _WROWS],
        out_shape=[
            jax.ShapeDtypeStruct((NPW, 128), f32),
            jax.ShapeDtypeStruct((NPW, 128), f32),
        ],
    )(s1, t1w, dga8, dgb8, e_mat, s_mat, gt, bt, cbt)

    t2 = t2w.reshape(NPAD, 16)
    s2 = _get_sc_round()(t2, src, dst)

    z = zw.reshape(NPAD, 16)[:NN]
    dom = pl.pallas_call(
        _dom_body,
        grid=(_GRID,),
        in_specs=[
            _rows(16),
            _full((16, 64)), _full((1, 64)),
            _full((64, 8)), _full((1, 8)),
        ],
        out_specs=_rows(8),
        out_shape=jax.ShapeDtypeStruct((NN, 8), f32),
    )(z, clf_W1, row(clf_b1), clf_W2, row(clf_b2))

    rec0, rec1 = pl.pallas_call(
        _dec_body,
        grid=(_GRID,),
        in_specs=[
            _WPART, _WROWS, _DEG8, _DEG8,
            _full((8, 128)),
            _full((128, 1024)), _full((1, 1024)),
            _full((128, 1024)), _full((1, 1024)),
        ],
        out_specs=[_rows(128), _rows(128)],
        out_shape=[
            jax.ShapeDtypeStruct((NN, 128), f32),
            jax.ShapeDtypeStruct((NN, 128), f32),
        ],
    )(s2, t2w, dga8, dgb8, e_mat, d0wb, d0bb, d1wb, d1bb)

    return z, rec0, rec1, dom
